# baseline (speedup 1.0000x reference)
import functools

import jax
import jax.numpy as jnp
import numpy as np
from jax import lax
from jax.experimental import pallas as pl
from jax.experimental.pallas import tpu as pltpu
from jax.experimental.pallas import tpu_sc as plsc

_NUM_CORES = 2
_NUM_SUBCORES = 16
_NW = _NUM_CORES * _NUM_SUBCORES
_CHUNK = 128
_LANES = 16


def _positional_encoding_np(max_len, dim):
    position = np.arange(max_len, dtype=np.float64)[:, None]
    i = np.arange(0, dim, 2, dtype=np.float64)[None, :] / dim
    exp_term = 10000.0 ** i
    enc = np.zeros((max_len, dim), dtype=np.float32)
    enc[:, 0::2] = np.sin(position / exp_term)
    enc[:, 1::2] = np.cos(position / exp_term)
    return enc


@functools.partial(jax.jit, static_argnames=("b", "l", "d"))
def _emb_lookup(x_flat, table, pe_ext_flat, *, b, l, d):
    n = b * l
    per_w = n // _NW
    chunks = per_w // _CHUNK
    pe_rows = l + _CHUNK
    mesh = plsc.VectorSubcoreMesh(
        core_axis_name="c", subcore_axis_name="s", num_cores=_NUM_CORES
    )

    @functools.partial(
        pl.kernel,
        out_type=jax.ShapeDtypeStruct((n, d), jnp.float32),
        mesh=mesh,
        scratch_types=[
            pltpu.VMEM((pe_rows * d,), jnp.float32),
            pltpu.VMEM((_CHUNK,), jnp.int32),
            pltpu.VMEM((_CHUNK, d), jnp.float32),
            pltpu.SemaphoreType.DMA,
        ],
        compiler_params=pltpu.CompilerParams(use_tc_tiling_on_sc=False),
    )
    def k(x_hbm, table_hbm, pe_hbm, out_hbm, pe_v, idx_v, rows_v, sem):
        wid = lax.axis_index("s") * _NUM_CORES + lax.axis_index("c")
        base = wid * per_w
        pltpu.sync_copy(pe_hbm, pe_v)

        @pl.loop(0, chunks)
        def _chunk(c):
            row0 = base + c * _CHUNK
            pltpu.sync_copy(x_hbm.at[pl.ds(row0, _CHUNK)], idx_v)
            pltpu.async_copy(table_hbm.at[idx_v], rows_v, sem).wait()
            p0 = lax.rem(c * _CHUNK, l)

            @pl.loop(0, _CHUNK)
            def _row(r):
                off = (p0 + r) * d
                for j in range(d // _LANES):
                    v = pe_v[pl.ds(off + j * _LANES, _LANES)]
                    plsc.addupdate(rows_v.at[r, pl.ds(j * _LANES, _LANES)], v)

            pltpu.sync_copy(rows_v, out_hbm.at[pl.ds(row0, _CHUNK)])

    return k(x_flat, table, pe_ext_flat)


def kernel(x, table):
    b, l = x.shape
    v, d = table.shape
    pe = _positional_encoding_np(l + _CHUNK, d)
    pe_ext = np.concatenate([pe[:l], pe[:_CHUNK]], axis=0)
    pe_ext_flat = jnp.asarray(pe_ext.reshape(-1))
    out = _emb_lookup(x.reshape(-1), table, pe_ext_flat, b=b, l=l, d=d)
    return out.reshape(b, l, d)


# speedup vs baseline: 1.4305x; 1.4305x over previous
import functools

import jax
import jax.numpy as jnp
import numpy as np
from jax import lax
from jax.experimental import pallas as pl
from jax.experimental.pallas import tpu as pltpu
from jax.experimental.pallas import tpu_sc as plsc

_NUM_CORES = 2
_NUM_SUBCORES = 16
_NW = _NUM_CORES * _NUM_SUBCORES
_LANES = 16
_SEQ_PER_CHUNK = 2
_SUBS = ((0, 104), (104, 104), (208, 96), (304, 96))


def _positional_encoding_np(max_len, dim):
    position = np.arange(max_len, dtype=np.float64)[:, None]
    i = np.arange(0, dim, 2, dtype=np.float64)[None, :] / dim
    exp_term = 10000.0 ** i
    enc = np.zeros((max_len, dim), dtype=np.float32)
    enc[:, 0::2] = np.sin(position / exp_term)
    enc[:, 1::2] = np.cos(position / exp_term)
    return enc


@functools.partial(jax.jit, static_argnames=("b", "l", "d"))
def _emb_lookup(x_flat, table, pe_flat, *, b, l, d):
    n = b * l
    per_w = n // _NW
    sc_rows = _SEQ_PER_CHUNK * l
    ng = per_w // sc_rows
    nvec = d // _LANES
    mesh = plsc.VectorSubcoreMesh(
        core_axis_name="c", subcore_axis_name="s", num_cores=_NUM_CORES
    )

    @functools.partial(
        pl.kernel,
        out_type=jax.ShapeDtypeStruct((n, d), jnp.float32),
        mesh=mesh,
        scratch_types=[
            pltpu.VMEM((l * d,), jnp.float32),
            pltpu.VMEM((sc_rows,), jnp.int32),
            pltpu.VMEM((sc_rows,), jnp.int32),
            pltpu.VMEM((2, sc_rows, d), jnp.float32),
            pltpu.SemaphoreType.DMA,
            pltpu.SemaphoreType.DMA,
            pltpu.SemaphoreType.DMA,
            pltpu.SemaphoreType.DMA,
        ],
        compiler_params=pltpu.CompilerParams(use_tc_tiling_on_sc=False),
    )
    def k(x_hbm, table_hbm, pe_hbm, out_hbm, pe_v, idx_v0, idx_v1, rows_v,
          gsem0, gsem1, ssem0, ssem1):
        gsem = (gsem0, gsem1)
        ssem = (ssem0, ssem1)
        idx_v = (idx_v0, idx_v1)
        wid = lax.axis_index("s") * _NUM_CORES + lax.axis_index("c")
        base = wid * per_w
        pltpu.sync_copy(pe_hbm, pe_v)

        def fire_gather(g, buf):
            row0 = base + g * sc_rows
            pltpu.sync_copy(x_hbm.at[pl.ds(row0, sc_rows)], idx_v[buf])
            for off, sz in _SUBS:
                pltpu.async_copy(
                    table_hbm.at[idx_v[buf].at[pl.ds(off, sz)]],
                    rows_v.at[buf, pl.ds(off, sz)],
                    gsem[buf],
                )

        def wait_gather(buf):
            pltpu.make_async_copy(
                out_hbm.at[pl.ds(0, sc_rows)], rows_v.at[buf], gsem[buf]
            ).wait()

        def wait_store(buf):
            pltpu.make_async_copy(
                rows_v.at[buf], out_hbm.at[pl.ds(0, sc_rows)], ssem[buf]
            ).wait()

        fire_gather(0, 0)

        @pl.loop(0, ng, step=2)
        def _super(g0):
            for bu in range(2):
                g = g0 + bu
                nb = 1 - bu
                wait_gather(bu)
                @pl.when(g + 1 < ng)
                def _():
                    @pl.when(g >= 1)
                    def _():
                        wait_store(nb)
                    fire_gather(g + 1, nb)

                for s in range(_SEQ_PER_CHUNK):
                    @pl.loop(0, l, unroll=4)
                    def _row(r):
                        for j in range(nvec):
                            v = pe_v[pl.ds(r * d + j * _LANES, _LANES)]
                            plsc.addupdate(
                                rows_v.at[bu, s * l + r,
                                          pl.ds(j * _LANES, _LANES)], v)

                pltpu.async_copy(
                    rows_v.at[bu],
                    out_hbm.at[pl.ds(base + g * sc_rows, sc_rows)],
                    ssem[bu],
                )

        wait_store(0)
        wait_store(1)

    return k(x_flat, table, pe_flat)


def kernel(x, table):
    b, l = x.shape
    v, d = table.shape
    pe = _positional_encoding_np(l, d)
    pe_flat = jnp.asarray(pe.reshape(-1))
    out = _emb_lookup(x.reshape(-1), table, pe_flat, b=b, l=l, d=d)
    return out.reshape(b, l, d)
